# Initial kernel scaffold; baseline (speedup 1.0000x reference)
import jax
import jax.numpy as jnp
from jax.experimental import pallas as pl


def kernel(x, edge_index, W_l1, b_l1, W_r1, W_l2, b_l2, W_r2, W_out, b_out):
    raise NotImplementedError("write your pallas kernel here")


# speedup vs baseline: 5.4341x; 5.4341x over previous
import jax
import jax.numpy as jnp
from jax import lax
from jax.experimental import pallas as pl
from jax.experimental.pallas import tpu as pltpu
from jax.experimental.pallas import tpu_sc as plsc

N = 10000
D = 128
E = 320000
NC = 2
NS = 16
NW = NC * NS
EPW = E // NW
K = 80
NCHUNK = EPW // K
RPT = 624
REM = N - NS * RPT
CW = 16
CFLAT = N * CW
CTILE = RPT * CW

_MESH = plsc.VectorSubcoreMesh(core_axis_name="c", subcore_axis_name="s",
                               num_cores=NC, num_subcores=NS)


def _sc_body(with_cnt, x_hbm, src_hbm, dst_hbm, zx_hbm, agg_out, cnt_out,
             idx_v, rows_v, sem, acc_sh, ones_v, cbuf, wbuf, cnt_sh):
    cid = lax.axis_index("c")
    sid = lax.axis_index("s")
    wid = sid * NC + cid
    row0 = sid * RPT
    last = sid == NS - 1

    pltpu.sync_copy(zx_hbm.at[pl.ds(row0, RPT)], acc_sh.at[pl.ds(row0, RPT)])

    @pl.when(last)
    def _():
        pltpu.sync_copy(zx_hbm.at[pl.ds(NS * RPT, REM)],
                        acc_sh.at[pl.ds(NS * RPT, REM)])

    if with_cnt:
        def fill_ones(i, c):
            ones_v[i] = jnp.ones((CW,), jnp.float32)
            return c

        lax.fori_loop(0, K, fill_ones, 0)

        def fill_zero(i, c):
            cbuf[i] = jnp.zeros((CW,), jnp.float32)
            return c

        lax.fori_loop(0, RPT, fill_zero, 0)
        pltpu.sync_copy(cbuf, cnt_sh.at[pl.ds(row0, RPT)])

        @pl.when(last)
        def _():
            pltpu.sync_copy(cbuf.at[pl.ds(0, REM)],
                            cnt_sh.at[pl.ds(NS * RPT, REM)])

    plsc.subcore_barrier()
    ebase = wid * EPW

    def chunk(c, carry):
        base = ebase + c * K
        pltpu.sync_copy(src_hbm.at[pl.ds(base, K)], idx_v.at[0])
        pltpu.sync_copy(dst_hbm.at[pl.ds(base, K)], idx_v.at[1])
        pltpu.async_copy(x_hbm.at[idx_v.at[0]], rows_v, sem).wait()
        pltpu.sync_copy(rows_v, acc_sh.at[idx_v.at[1]], add=True)
        if with_cnt:
            pltpu.sync_copy(ones_v, cnt_sh.at[idx_v.at[1]], add=True)
        return carry

    lax.fori_loop(0, NCHUNK, chunk, 0)
    plsc.subcore_barrier()

    pltpu.sync_copy(acc_sh.at[pl.ds(row0, RPT)],
                    agg_out.at[cid, pl.ds(row0, RPT)])

    @pl.when(last)
    def _():
        pltpu.sync_copy(acc_sh.at[pl.ds(NS * RPT, REM)],
                        agg_out.at[cid, pl.ds(NS * RPT, REM)])

    if with_cnt:
        pltpu.sync_copy(cnt_sh.at[pl.ds(row0, RPT)], cbuf)

        def flat(i, c):
            wbuf[pl.ds(i * CW, CW)] = cbuf[i]
            return c

        lax.fori_loop(0, RPT, flat, 0)
        pltpu.sync_copy(wbuf, cnt_out.at[cid, pl.ds(sid * CTILE, CTILE)])

        @pl.when(last)
        def _():
            pltpu.sync_copy(cnt_sh.at[pl.ds(NS * RPT, REM)],
                            cbuf.at[pl.ds(0, REM)])

            def flat2(i, c):
                wbuf[pl.ds(i * CW, CW)] = cbuf[i]
                return c

            lax.fori_loop(0, REM, flat2, 0)
            pltpu.sync_copy(wbuf.at[pl.ds(0, REM * CW)],
                            cnt_out.at[cid, pl.ds(NS * CTILE, REM * CW)])


def _make_sc_agg(with_cnt: bool):
    out_type = [jax.ShapeDtypeStruct((NC, N, D), jnp.float32)]
    scratch = [
        pltpu.VMEM((2, K), jnp.int32),
        pltpu.VMEM((K, D), jnp.float32),
        pltpu.SemaphoreType.DMA,
        pltpu.VMEM_SHARED((N, D), jnp.float32),
    ]
    if with_cnt:
        out_type.append(jax.ShapeDtypeStruct((NC, CFLAT), jnp.float32))
        scratch += [
            pltpu.VMEM((K, CW), jnp.float32),
            pltpu.VMEM((RPT, CW), jnp.float32),
            pltpu.VMEM((CTILE,), jnp.float32),
            pltpu.VMEM_SHARED((N, CW), jnp.float32),
        ]

        def body(x_hbm, src_hbm, dst_hbm, zx_hbm, agg_out, cnt_out,
                 idx_v, rows_v, sem, acc_sh, ones_v, cbuf, wbuf, cnt_sh):
            _sc_body(True, x_hbm, src_hbm, dst_hbm, zx_hbm, agg_out, cnt_out,
                     idx_v, rows_v, sem, acc_sh, ones_v, cbuf, wbuf, cnt_sh)
    else:
        def body(x_hbm, src_hbm, dst_hbm, zx_hbm, agg_out,
                 idx_v, rows_v, sem, acc_sh):
            _sc_body(False, x_hbm, src_hbm, dst_hbm, zx_hbm, agg_out, None,
                     idx_v, rows_v, sem, acc_sh, None, None, None, None)

    return pl.kernel(body, out_type=out_type, mesh=_MESH,
                     scratch_types=scratch,
                     compiler_params=pltpu.CompilerParams(
                         use_tc_tiling_on_sc=False))


_sc_agg_cnt = _make_sc_agg(True)
_sc_agg = _make_sc_agg(False)

BN = 1000


def _tc1_body(agg_ref, cnt_ref, x_ref, wl_ref, bl_ref, wr_ref, o_ref):
    agg = agg_ref[0] + agg_ref[1]
    cnt = cnt_ref[0, :, 0:1] + cnt_ref[1, :, 0:1]
    mean = agg / jnp.maximum(cnt, 1.0)
    h = (jnp.dot(mean, wl_ref[...], preferred_element_type=jnp.float32)
         + bl_ref[...]
         + jnp.dot(x_ref[...], wr_ref[...], preferred_element_type=jnp.float32))
    o_ref[...] = jnp.maximum(h, 0.0)


def _tc2_body(agg_ref, cnt_ref, h_ref, wl_ref, bl_ref, wr_ref,
              wo_ref, bo_ref, o_ref):
    agg = agg_ref[0] + agg_ref[1]
    cnt = cnt_ref[0, :, 0:1] + cnt_ref[1, :, 0:1]
    mean = agg / jnp.maximum(cnt, 1.0)
    h = (jnp.dot(mean, wl_ref[...], preferred_element_type=jnp.float32)
         + bl_ref[...]
         + jnp.dot(h_ref[...], wr_ref[...], preferred_element_type=jnp.float32))
    h = jnp.maximum(h, 0.0)
    o_ref[...] = (jnp.dot(h, wo_ref[...], preferred_element_type=jnp.float32)
                  + bo_ref[...])


def _full_spec(shape):
    return pl.BlockSpec(shape, lambda i: tuple(0 for _ in shape))


_AGG_SPEC = pl.BlockSpec((NC, BN, D), lambda i: (0, i, 0))
_CNT_SPEC = pl.BlockSpec((NC, BN, CW), lambda i: (0, i, 0))
_X_SPEC = pl.BlockSpec((BN, D), lambda i: (i, 0))

_tc1 = pl.pallas_call(
    _tc1_body,
    grid=(N // BN,),
    in_specs=[_AGG_SPEC, _CNT_SPEC, _X_SPEC,
              _full_spec((D, D)), _full_spec((1, D)), _full_spec((D, D))],
    out_specs=_X_SPEC,
    out_shape=jax.ShapeDtypeStruct((N, D), jnp.float32),
)

_tc2 = pl.pallas_call(
    _tc2_body,
    grid=(N // BN,),
    in_specs=[_AGG_SPEC, _CNT_SPEC, _X_SPEC,
              _full_spec((D, D)), _full_spec((1, D)), _full_spec((D, D)),
              _full_spec((D, D)), _full_spec((1, D))],
    out_specs=_X_SPEC,
    out_shape=jax.ShapeDtypeStruct((N, D), jnp.float32),
)


@jax.jit
def kernel(x, edge_index, W_l1, b_l1, W_r1, W_l2, b_l2, W_r2, W_out, b_out):
    src = edge_index[0].astype(jnp.int32)
    dst = edge_index[1].astype(jnp.int32)
    zx = jnp.zeros((N, D), jnp.float32)
    agg1, cnt = _sc_agg_cnt(x, src, dst, zx)
    cnt = cnt.reshape(NC, N, CW)
    h1 = _tc1(agg1, cnt, x, W_l1, b_l1.reshape(1, D), W_r1)
    agg2 = _sc_agg(h1, src, dst, zx)[0]
    return _tc2(agg2, cnt, h1, W_l2, b_l2.reshape(1, D), W_r2,
                W_out, b_out.reshape(1, D))


# speedup vs baseline: 9.6225x; 1.7708x over previous
import jax
import jax.numpy as jnp
from jax import lax
from jax.experimental import pallas as pl
from jax.experimental.pallas import tpu as pltpu
from jax.experimental.pallas import tpu_sc as plsc

N = 10000
D = 128
E = 320000
NC = 2
NS = 16
NW = NC * NS
EPW = E // NW
K = 80
NCHUNK = EPW // K
RPT = 624
REM = N - NS * RPT
CW = 16
CFLAT = N * CW
CTILE = RPT * CW
HRPT = RPT // 2

_MESH = plsc.VectorSubcoreMesh(core_axis_name="c", subcore_axis_name="s",
                               num_cores=NC, num_subcores=NS)


def _sc_body(with_cnt, x_hbm, eidx_hbm, zx_hbm, agg_out, cnt_out,
             idx_v, rows_v, isem, gsem0, gsem1, acc_sh, ones_v, cbuf, wbuf,
             cnt_sh):
    cid = lax.axis_index("c")
    sid = lax.axis_index("s")
    wid = sid * NC + cid
    row0 = sid * RPT
    last = sid == NS - 1
    gsem = (gsem0, gsem1)

    pltpu.sync_copy(zx_hbm.at[pl.ds(row0, RPT)], acc_sh.at[pl.ds(row0, RPT)])

    @pl.when(last)
    def _():
        pltpu.sync_copy(zx_hbm.at[pl.ds(NS * RPT, REM)],
                        acc_sh.at[pl.ds(NS * RPT, REM)])

    if with_cnt:
        def fill_ones(i, c):
            ones_v[i] = jnp.ones((CW,), jnp.float32)
            return c

        lax.fori_loop(0, K, fill_ones, 0)

        def fill_zero(i, c):
            cbuf[i] = jnp.zeros((CW,), jnp.float32)
            return c

        lax.fori_loop(0, HRPT, fill_zero, 0)
        pltpu.sync_copy(cbuf, cnt_sh.at[pl.ds(row0, HRPT)])
        pltpu.sync_copy(cbuf, cnt_sh.at[pl.ds(row0 + HRPT, HRPT)])

        @pl.when(last)
        def _():
            pltpu.sync_copy(cbuf.at[pl.ds(0, REM)],
                            cnt_sh.at[pl.ds(NS * RPT, REM)])

    plsc.subcore_barrier()

    def scatter(b):
        pltpu.sync_copy(rows_v.at[b], acc_sh.at[idx_v.at[b, 1]], add=True)
        if with_cnt:
            pltpu.sync_copy(ones_v, cnt_sh.at[idx_v.at[b, 1]], add=True)

    pltpu.sync_copy(eidx_hbm.at[wid, 0], idx_v.at[0])
    pltpu.async_copy(x_hbm.at[idx_v.at[0, 0]], rows_v.at[0], gsem[0])
    pltpu.async_copy(eidx_hbm.at[wid, 1], idx_v.at[1], isem)

    def pair(i, carry):
        c = 2 * i
        for b in (0, 1):
            nb = 1 - b
            cc = c + b
            pltpu.make_async_copy(eidx_hbm.at[wid, 0], idx_v.at[nb],
                                  isem).wait()
            pltpu.async_copy(x_hbm.at[idx_v.at[nb, 0]], rows_v.at[nb],
                             gsem[nb])
            pltpu.make_async_copy(x_hbm.at[idx_v.at[b, 0]], rows_v.at[b],
                                  gsem[b]).wait()
            scatter(b)
            nidx = jnp.minimum(cc + 2, NCHUNK - 1)
            pltpu.async_copy(eidx_hbm.at[wid, nidx], idx_v.at[b], isem)
        return carry

    lax.fori_loop(0, NCHUNK // 2, pair, 0)
    pltpu.make_async_copy(eidx_hbm.at[wid, 0], idx_v.at[1], isem).wait()
    pltpu.make_async_copy(x_hbm.at[idx_v.at[0, 0]], rows_v.at[0],
                          gsem[0]).wait()
    scatter(0)
    plsc.subcore_barrier()

    pltpu.sync_copy(acc_sh.at[pl.ds(row0, RPT)],
                    agg_out.at[cid, pl.ds(row0, RPT)])

    @pl.when(last)
    def _():
        pltpu.sync_copy(acc_sh.at[pl.ds(NS * RPT, REM)],
                        agg_out.at[cid, pl.ds(NS * RPT, REM)])

    if with_cnt:
        def flat(i, c):
            wbuf[pl.ds(i * CW, CW)] = cbuf[i]
            return c

        for h in (0, 1):
            pltpu.sync_copy(cnt_sh.at[pl.ds(row0 + h * HRPT, HRPT)], cbuf)
            lax.fori_loop(0, HRPT, flat, 0)
            pltpu.sync_copy(
                wbuf,
                cnt_out.at[cid, pl.ds(sid * CTILE + h * HRPT * CW,
                                      HRPT * CW)])

        @pl.when(last)
        def _():
            pltpu.sync_copy(cnt_sh.at[pl.ds(NS * RPT, REM)],
                            cbuf.at[pl.ds(0, REM)])
            lax.fori_loop(0, REM, flat, 0)
            pltpu.sync_copy(wbuf.at[pl.ds(0, REM * CW)],
                            cnt_out.at[cid, pl.ds(NS * CTILE, REM * CW)])


def _make_sc_agg(with_cnt: bool):
    out_type = [jax.ShapeDtypeStruct((NC, N, D), jnp.float32)]
    scratch = [
        pltpu.VMEM((2, 2, K), jnp.int32),
        pltpu.VMEM((2, K, D), jnp.float32),
        pltpu.SemaphoreType.DMA,
        pltpu.SemaphoreType.DMA,
        pltpu.SemaphoreType.DMA,
        pltpu.VMEM_SHARED((N, D), jnp.float32),
    ]
    if with_cnt:
        out_type.append(jax.ShapeDtypeStruct((NC, CFLAT), jnp.float32))
        scratch += [
            pltpu.VMEM((K, CW), jnp.float32),
            pltpu.VMEM((HRPT, CW), jnp.float32),
            pltpu.VMEM((HRPT * CW,), jnp.float32),
            pltpu.VMEM_SHARED((N, CW), jnp.float32),
        ]

        def body(x_hbm, eidx_hbm, zx_hbm, agg_out, cnt_out,
                 idx_v, rows_v, isem, gsem0, gsem1, acc_sh, ones_v, cbuf,
                 wbuf, cnt_sh):
            _sc_body(True, x_hbm, eidx_hbm, zx_hbm, agg_out, cnt_out,
                     idx_v, rows_v, isem, gsem0, gsem1, acc_sh, ones_v, cbuf,
                     wbuf, cnt_sh)
    else:
        def body(x_hbm, eidx_hbm, zx_hbm, agg_out,
                 idx_v, rows_v, isem, gsem0, gsem1, acc_sh):
            _sc_body(False, x_hbm, eidx_hbm, zx_hbm, agg_out, None,
                     idx_v, rows_v, isem, gsem0, gsem1, acc_sh, None, None,
                     None, None)

    return pl.kernel(body, out_type=out_type, mesh=_MESH,
                     scratch_types=scratch,
                     compiler_params=pltpu.CompilerParams(
                         use_tc_tiling_on_sc=False))


_sc_agg_cnt = _make_sc_agg(True)
_sc_agg = _make_sc_agg(False)

BN = 1000


def _tc1_body(agg_ref, cnt_ref, x_ref, wl_ref, bl_ref, wr_ref, o_ref):
    agg = agg_ref[0] + agg_ref[1]
    cnt = cnt_ref[0, :, 0:1] + cnt_ref[1, :, 0:1]
    mean = agg / jnp.maximum(cnt, 1.0)
    h = (jnp.dot(mean, wl_ref[...], preferred_element_type=jnp.float32)
         + bl_ref[...]
         + jnp.dot(x_ref[...], wr_ref[...], preferred_element_type=jnp.float32))
    o_ref[...] = jnp.maximum(h, 0.0)


def _tc2_body(agg_ref, cnt_ref, h_ref, wl_ref, bl_ref, wr_ref,
              wo_ref, bo_ref, o_ref):
    agg = agg_ref[0] + agg_ref[1]
    cnt = cnt_ref[0, :, 0:1] + cnt_ref[1, :, 0:1]
    mean = agg / jnp.maximum(cnt, 1.0)
    h = (jnp.dot(mean, wl_ref[...], preferred_element_type=jnp.float32)
         + bl_ref[...]
         + jnp.dot(h_ref[...], wr_ref[...], preferred_element_type=jnp.float32))
    h = jnp.maximum(h, 0.0)
    o_ref[...] = (jnp.dot(h, wo_ref[...], preferred_element_type=jnp.float32)
                  + bo_ref[...])


def _full_spec(shape):
    return pl.BlockSpec(shape, lambda i: tuple(0 for _ in shape))


_AGG_SPEC = pl.BlockSpec((NC, BN, D), lambda i: (0, i, 0))
_CNT_SPEC = pl.BlockSpec((NC, BN, CW), lambda i: (0, i, 0))
_X_SPEC = pl.BlockSpec((BN, D), lambda i: (i, 0))

_tc1 = pl.pallas_call(
    _tc1_body,
    grid=(N // BN,),
    in_specs=[_AGG_SPEC, _CNT_SPEC, _X_SPEC,
              _full_spec((D, D)), _full_spec((1, D)), _full_spec((D, D))],
    out_specs=_X_SPEC,
    out_shape=jax.ShapeDtypeStruct((N, D), jnp.float32),
)

_tc2 = pl.pallas_call(
    _tc2_body,
    grid=(N // BN,),
    in_specs=[_AGG_SPEC, _CNT_SPEC, _X_SPEC,
              _full_spec((D, D)), _full_spec((1, D)), _full_spec((D, D)),
              _full_spec((D, D)), _full_spec((1, D))],
    out_specs=_X_SPEC,
    out_shape=jax.ShapeDtypeStruct((N, D), jnp.float32),
)


@jax.jit
def kernel(x, edge_index, W_l1, b_l1, W_r1, W_l2, b_l2, W_r2, W_out, b_out):
    src = edge_index[0].astype(jnp.int32)
    dst = edge_index[1].astype(jnp.int32)
    eidx = jnp.stack([src.reshape(NW, NCHUNK, K),
                      dst.reshape(NW, NCHUNK, K)], axis=2)
    zx = jnp.zeros((N, D), jnp.float32)
    agg1, cnt = _sc_agg_cnt(x, eidx, zx)
    cnt = cnt.reshape(NC, N, CW)
    h1 = _tc1(agg1, cnt, x, W_l1, b_l1.reshape(1, D), W_r1)
    agg2 = _sc_agg(h1, eidx, zx)[0]
    return _tc2(agg2, cnt, h1, W_l2, b_l2.reshape(1, D), W_r2,
                W_out, b_out.reshape(1, D))


# speedup vs baseline: 11.8807x; 1.2347x over previous
import jax
import jax.numpy as jnp
from jax import lax
from jax.experimental import pallas as pl
from jax.experimental.pallas import tpu as pltpu
from jax.experimental.pallas import tpu_sc as plsc

N = 10000
D = 128
E = 320000
NC = 2
NS = 16
NW = NC * NS
EPW = E // NW
RPT = 624
REM = N - NS * RPT
CW = 16
CFLAT = N * CW
CTILE = RPT * CW
HRPT = RPT // 4
KT = 16

_MESH = plsc.VectorSubcoreMesh(core_axis_name="c", subcore_axis_name="s",
                               num_cores=NC, num_subcores=NS)


def _sc_body(with_cnt, K, x_hbm, eidx_hbm, zx_hbm, agg_out, cnt_out,
             idx_v, rows_v, tidx_v, isem, gsem0, gsem1, acc_sh, ones_v, cbuf,
             wbuf, cnt_sh):
    NCHUNK = (EPW - KT) // K
    cid = lax.axis_index("c")
    sid = lax.axis_index("s")
    wid = sid * NC + cid
    row0 = sid * RPT
    last = sid == NS - 1
    gsem = (gsem0, gsem1)

    pltpu.sync_copy(zx_hbm.at[pl.ds(row0, RPT)], acc_sh.at[pl.ds(row0, RPT)])

    @pl.when(last)
    def _():
        pltpu.sync_copy(zx_hbm.at[pl.ds(NS * RPT, REM)],
                        acc_sh.at[pl.ds(NS * RPT, REM)])

    if with_cnt:
        def fill_ones(i, c):
            ones_v[i] = jnp.ones((CW,), jnp.float32)
            return c

        lax.fori_loop(0, K, fill_ones, 0)

        def fill_zero(i, c):
            cbuf[i] = jnp.zeros((CW,), jnp.float32)
            return c

        lax.fori_loop(0, HRPT, fill_zero, 0)
        for h in range(4):
            pltpu.sync_copy(cbuf, cnt_sh.at[pl.ds(row0 + h * HRPT, HRPT)])

        @pl.when(last)
        def _():
            pltpu.sync_copy(cbuf.at[pl.ds(0, REM)],
                            cnt_sh.at[pl.ds(NS * RPT, REM)])

    plsc.subcore_barrier()
    ebase = wid * EPW

    def scatter(b):
        pltpu.sync_copy(rows_v.at[b], acc_sh.at[idx_v.at[b, 1]], add=True)
        if with_cnt:
            pltpu.sync_copy(ones_v, cnt_sh.at[idx_v.at[b, 1]], add=True)

    def idx_slice(c):
        return eidx_hbm.at[:, pl.ds(ebase + c * K, K)]

    pltpu.sync_copy(idx_slice(0), idx_v.at[0])
    pltpu.async_copy(x_hbm.at[idx_v.at[0, 0]], rows_v.at[0], gsem[0])
    pltpu.async_copy(idx_slice(1), idx_v.at[1], isem)

    def pair(i, carry):
        c = 2 * i
        for b in (0, 1):
            nb = 1 - b
            cc = c + b
            pltpu.make_async_copy(idx_slice(0), idx_v.at[nb], isem).wait()
            pltpu.async_copy(x_hbm.at[idx_v.at[nb, 0]], rows_v.at[nb],
                             gsem[nb])
            pltpu.make_async_copy(x_hbm.at[idx_v.at[b, 0]], rows_v.at[b],
                                  gsem[b]).wait()
            scatter(b)
            nidx = jnp.minimum(cc + 2, NCHUNK - 1)
            pltpu.async_copy(idx_slice(nidx), idx_v.at[b], isem)
        return carry

    lax.fori_loop(0, NCHUNK // 2 - 1, pair, 0)
    pltpu.make_async_copy(idx_slice(0), idx_v.at[1], isem).wait()
    pltpu.async_copy(x_hbm.at[idx_v.at[1, 0]], rows_v.at[1], gsem[1])
    pltpu.make_async_copy(x_hbm.at[idx_v.at[0, 0]], rows_v.at[0],
                          gsem[0]).wait()
    scatter(0)
    pltpu.make_async_copy(x_hbm.at[idx_v.at[1, 0]], rows_v.at[1],
                          gsem[1]).wait()
    scatter(1)
    pltpu.sync_copy(eidx_hbm.at[:, pl.ds(ebase + NCHUNK * K, KT)], tidx_v)
    pltpu.async_copy(x_hbm.at[tidx_v.at[0]], rows_v.at[0, pl.ds(0, KT)],
                     gsem[0]).wait()
    pltpu.sync_copy(rows_v.at[0, pl.ds(0, KT)], acc_sh.at[tidx_v.at[1]],
                    add=True)
    if with_cnt:
        pltpu.sync_copy(ones_v.at[pl.ds(0, KT)], cnt_sh.at[tidx_v.at[1]],
                        add=True)
    plsc.subcore_barrier()

    pltpu.sync_copy(acc_sh.at[pl.ds(row0, RPT)],
                    agg_out.at[cid, pl.ds(row0, RPT)])

    @pl.when(last)
    def _():
        pltpu.sync_copy(acc_sh.at[pl.ds(NS * RPT, REM)],
                        agg_out.at[cid, pl.ds(NS * RPT, REM)])

    if with_cnt:
        def flat(i, c):
            wbuf[pl.ds(i * CW, CW)] = cbuf[i]
            return c

        for h in range(4):
            pltpu.sync_copy(cnt_sh.at[pl.ds(row0 + h * HRPT, HRPT)], cbuf)
            lax.fori_loop(0, HRPT, flat, 0)
            pltpu.sync_copy(
                wbuf,
                cnt_out.at[cid, pl.ds(sid * CTILE + h * HRPT * CW,
                                      HRPT * CW)])

        @pl.when(last)
        def _():
            pltpu.sync_copy(cnt_sh.at[pl.ds(NS * RPT, REM)],
                            cbuf.at[pl.ds(0, REM)])
            lax.fori_loop(0, REM, flat, 0)
            pltpu.sync_copy(wbuf.at[pl.ds(0, REM * CW)],
                            cnt_out.at[cid, pl.ds(NS * CTILE, REM * CW)])


def _make_sc_agg(with_cnt: bool, K: int):
    out_type = [jax.ShapeDtypeStruct((NC, N, D), jnp.float32)]
    scratch = [
        pltpu.VMEM((2, 2, K), jnp.int32),
        pltpu.VMEM((2, K, D), jnp.float32),
        pltpu.VMEM((2, KT), jnp.int32),
        pltpu.SemaphoreType.DMA,
        pltpu.SemaphoreType.DMA,
        pltpu.SemaphoreType.DMA,
        pltpu.VMEM_SHARED((N, D), jnp.float32),
    ]
    if with_cnt:
        out_type.append(jax.ShapeDtypeStruct((NC, CFLAT), jnp.float32))
        scratch += [
            pltpu.VMEM((K, CW), jnp.float32),
            pltpu.VMEM((HRPT, CW), jnp.float32),
            pltpu.VMEM((HRPT * CW,), jnp.float32),
            pltpu.VMEM_SHARED((N, CW), jnp.float32),
        ]

        def body(x_hbm, eidx_hbm, zx_hbm, agg_out, cnt_out,
                 idx_v, rows_v, tidx_v, isem, gsem0, gsem1, acc_sh, ones_v,
                 cbuf, wbuf, cnt_sh):
            _sc_body(True, K, x_hbm, eidx_hbm, zx_hbm, agg_out, cnt_out,
                     idx_v, rows_v, tidx_v, isem, gsem0, gsem1, acc_sh,
                     ones_v, cbuf, wbuf, cnt_sh)
    else:
        def body(x_hbm, eidx_hbm, zx_hbm, agg_out,
                 idx_v, rows_v, tidx_v, isem, gsem0, gsem1, acc_sh):
            _sc_body(False, K, x_hbm, eidx_hbm, zx_hbm, agg_out, None,
                     idx_v, rows_v, tidx_v, isem, gsem0, gsem1, acc_sh,
                     None, None, None, None)

    return pl.kernel(body, out_type=out_type, mesh=_MESH,
                     scratch_types=scratch,
                     compiler_params=pltpu.CompilerParams(
                         use_tc_tiling_on_sc=False))


_sc_agg_cnt = _make_sc_agg(True, 104)
_sc_agg = _make_sc_agg(False, 128)

BN = 1000


def _tc1_body(agg_ref, cnt_ref, x_ref, wl_ref, bl_ref, wr_ref, o_ref):
    agg = agg_ref[0] + agg_ref[1]
    cnt = cnt_ref[0, :, 0:1] + cnt_ref[1, :, 0:1]
    mean = agg / jnp.maximum(cnt, 1.0)
    h = (jnp.dot(mean, wl_ref[...], preferred_element_type=jnp.float32)
         + bl_ref[...]
         + jnp.dot(x_ref[...], wr_ref[...], preferred_element_type=jnp.float32))
    o_ref[...] = jnp.maximum(h, 0.0)


def _tc2_body(agg_ref, cnt_ref, h_ref, wl_ref, bl_ref, wr_ref,
              wo_ref, bo_ref, o_ref):
    agg = agg_ref[0] + agg_ref[1]
    cnt = cnt_ref[0, :, 0:1] + cnt_ref[1, :, 0:1]
    mean = agg / jnp.maximum(cnt, 1.0)
    h = (jnp.dot(mean, wl_ref[...], preferred_element_type=jnp.float32)
         + bl_ref[...]
         + jnp.dot(h_ref[...], wr_ref[...], preferred_element_type=jnp.float32))
    h = jnp.maximum(h, 0.0)
    o_ref[...] = (jnp.dot(h, wo_ref[...], preferred_element_type=jnp.float32)
                  + bo_ref[...])


def _full_spec(shape):
    return pl.BlockSpec(shape, lambda i: tuple(0 for _ in shape))


_AGG_SPEC = pl.BlockSpec((NC, BN, D), lambda i: (0, i, 0))
_CNT_SPEC = pl.BlockSpec((NC, BN, CW), lambda i: (0, i, 0))
_X_SPEC = pl.BlockSpec((BN, D), lambda i: (i, 0))

_tc1 = pl.pallas_call(
    _tc1_body,
    grid=(N // BN,),
    in_specs=[_AGG_SPEC, _CNT_SPEC, _X_SPEC,
              _full_spec((D, D)), _full_spec((1, D)), _full_spec((D, D))],
    out_specs=_X_SPEC,
    out_shape=jax.ShapeDtypeStruct((N, D), jnp.float32),
)

_tc2 = pl.pallas_call(
    _tc2_body,
    grid=(N // BN,),
    in_specs=[_AGG_SPEC, _CNT_SPEC, _X_SPEC,
              _full_spec((D, D)), _full_spec((1, D)), _full_spec((D, D)),
              _full_spec((D, D)), _full_spec((1, D))],
    out_specs=_X_SPEC,
    out_shape=jax.ShapeDtypeStruct((N, D), jnp.float32),
)


@jax.jit
def kernel(x, edge_index, W_l1, b_l1, W_r1, W_l2, b_l2, W_r2, W_out, b_out):
    eidx = edge_index.astype(jnp.int32)
    zx = jnp.zeros((N, D), jnp.float32)
    agg1, cnt = _sc_agg_cnt(x, eidx, zx)
    cnt = cnt.reshape(NC, N, CW)
    h1 = _tc1(agg1, cnt, x, W_l1, b_l1.reshape(1, D), W_r1)
    agg2 = _sc_agg(h1, eidx, zx)[0]
    return _tc2(agg2, cnt, h1, W_l2, b_l2.reshape(1, D), W_r2,
                W_out, b_out.reshape(1, D))


# speedup vs baseline: 13.2582x; 1.1159x over previous
import jax
import jax.numpy as jnp
from jax import lax
from jax.experimental import pallas as pl
from jax.experimental.pallas import tpu as pltpu
from jax.experimental.pallas import tpu_sc as plsc

N = 10000
D = 128
E = 320000
NC = 2
NS = 16
NW = NC * NS
EPW = E // NW
RPT = 624
REM = N - NS * RPT
CW = 16
CFLAT = N * CW
CTILE = RPT * CW
HRPT = RPT // 4
KT = 16

_MESH = plsc.VectorSubcoreMesh(core_axis_name="c", subcore_axis_name="s",
                               num_cores=NC, num_subcores=NS)


def _sc_body(with_cnt, K, x_hbm, eidx_hbm, zx_hbm, agg_out, cnt_out,
             idx_v, rows_v, tidx_v, isem, gsem0, gsem1, ssem, csem, acc_sh,
             ones_v, cbuf, wbuf, cnt_sh):
    NCHUNK = (EPW - KT) // K
    cid = lax.axis_index("c")
    sid = lax.axis_index("s")
    wid = sid * NC + cid
    row0 = sid * RPT
    last = sid == NS - 1
    gsem = (gsem0, gsem1)

    pltpu.sync_copy(zx_hbm.at[pl.ds(row0, RPT)], acc_sh.at[pl.ds(row0, RPT)])

    @pl.when(last)
    def _():
        pltpu.sync_copy(zx_hbm.at[pl.ds(NS * RPT, REM)],
                        acc_sh.at[pl.ds(NS * RPT, REM)])

    if with_cnt:
        def fill_ones(i, c):
            ones_v[i] = jnp.ones((CW,), jnp.float32)
            return c

        lax.fori_loop(0, K, fill_ones, 0)

        def fill_zero(i, c):
            cbuf[i] = jnp.zeros((CW,), jnp.float32)
            return c

        lax.fori_loop(0, HRPT, fill_zero, 0)
        for h in range(4):
            pltpu.sync_copy(cbuf, cnt_sh.at[pl.ds(row0 + h * HRPT, HRPT)])

        @pl.when(last)
        def _():
            pltpu.sync_copy(cbuf.at[pl.ds(0, REM)],
                            cnt_sh.at[pl.ds(NS * RPT, REM)])

    plsc.subcore_barrier()
    ebase = wid * EPW

    def idx_slice(c):
        return eidx_hbm.at[:, pl.ds(ebase + c * K, K)]

    def scatter_start(b, r):
        pltpu.async_copy(rows_v.at[b], acc_sh.at[idx_v.at[r, 1]], ssem,
                         add=True)
        if with_cnt:
            pltpu.async_copy(ones_v, cnt_sh.at[idx_v.at[r, 1]], csem,
                             add=True)

    def scatter_wait(b, r):
        pltpu.make_async_copy(rows_v.at[b], acc_sh.at[idx_v.at[r, 1]],
                              ssem).wait()
        if with_cnt:
            pltpu.make_async_copy(ones_v, cnt_sh.at[idx_v.at[r, 1]],
                                  csem).wait()

    def gather_start(b, r):
        pltpu.async_copy(x_hbm.at[idx_v.at[r, 0]], rows_v.at[b], gsem[b])

    def gather_wait(b, r):
        pltpu.make_async_copy(x_hbm.at[idx_v.at[r, 0]], rows_v.at[b],
                              gsem[b]).wait()

    def idx_wait(r):
        pltpu.make_async_copy(idx_slice(0), idx_v.at[r], isem).wait()

    pltpu.sync_copy(idx_slice(0), idx_v.at[0])
    gather_start(0, 0)
    pltpu.async_copy(idx_slice(1), idx_v.at[1], isem)
    idx_wait(1)
    gather_start(1, 1)
    pltpu.async_copy(idx_slice(2), idx_v.at[2], isem)
    gather_wait(0, 0)
    scatter_start(0, 0)
    idx_wait(2)
    scatter_wait(0, 0)
    gather_start(0, 2)
    pltpu.async_copy(idx_slice(3), idx_v.at[3], isem)
    gather_wait(1, 1)
    scatter_start(1, 1)

    def quad(i, carry):
        c = 2 + 4 * i
        for j in range(4):
            cc = c + j
            b = (2 + j) % 2
            r = (2 + j) % 4
            nb = 1 - b
            nr = (r + 1) % 4
            idx_wait(nr)
            scatter_wait(nb, (r + 3) % 4)
            gather_start(nb, nr)
            pltpu.async_copy(idx_slice(cc + 2), idx_v.at[(r + 2) % 4], isem)
            gather_wait(b, r)
            scatter_start(b, r)
        return carry

    lax.fori_loop(0, (NCHUNK - 4) // 4, quad, 0)
    idx_wait(3)
    scatter_wait(1, 1)
    gather_start(1, 3)
    gather_wait(0, 2)
    scatter_start(0, 2)
    scatter_wait(0, 2)
    gather_wait(1, 3)
    scatter_start(1, 3)
    scatter_wait(1, 3)
    pltpu.sync_copy(eidx_hbm.at[:, pl.ds(ebase + NCHUNK * K, KT)], tidx_v)
    pltpu.async_copy(x_hbm.at[tidx_v.at[0]], rows_v.at[0, pl.ds(0, KT)],
                     gsem[0]).wait()
    pltpu.sync_copy(rows_v.at[0, pl.ds(0, KT)], acc_sh.at[tidx_v.at[1]],
                    add=True)
    if with_cnt:
        pltpu.sync_copy(ones_v.at[pl.ds(0, KT)], cnt_sh.at[tidx_v.at[1]],
                        add=True)
    plsc.subcore_barrier()

    pltpu.sync_copy(acc_sh.at[pl.ds(row0, RPT)],
                    agg_out.at[cid, pl.ds(row0, RPT)])

    @pl.when(last)
    def _():
        pltpu.sync_copy(acc_sh.at[pl.ds(NS * RPT, REM)],
                        agg_out.at[cid, pl.ds(NS * RPT, REM)])

    if with_cnt:
        def flat(i, c):
            wbuf[pl.ds(i * CW, CW)] = cbuf[i]
            return c

        for h in range(4):
            pltpu.sync_copy(cnt_sh.at[pl.ds(row0 + h * HRPT, HRPT)], cbuf)
            lax.fori_loop(0, HRPT, flat, 0)
            pltpu.sync_copy(
                wbuf,
                cnt_out.at[cid, pl.ds(sid * CTILE + h * HRPT * CW,
                                      HRPT * CW)])

        @pl.when(last)
        def _():
            pltpu.sync_copy(cnt_sh.at[pl.ds(NS * RPT, REM)],
                            cbuf.at[pl.ds(0, REM)])
            lax.fori_loop(0, REM, flat, 0)
            pltpu.sync_copy(wbuf.at[pl.ds(0, REM * CW)],
                            cnt_out.at[cid, pl.ds(NS * CTILE, REM * CW)])


def _make_sc_agg(with_cnt: bool, K: int):
    out_type = [jax.ShapeDtypeStruct((NC, N, D), jnp.float32)]
    scratch = [
        pltpu.VMEM((4, 2, K), jnp.int32),
        pltpu.VMEM((2, K, D), jnp.float32),
        pltpu.VMEM((2, KT), jnp.int32),
        pltpu.SemaphoreType.DMA,
        pltpu.SemaphoreType.DMA,
        pltpu.SemaphoreType.DMA,
        pltpu.SemaphoreType.DMA,
        pltpu.SemaphoreType.DMA,
        pltpu.VMEM_SHARED((N, D), jnp.float32),
    ]
    if with_cnt:
        out_type.append(jax.ShapeDtypeStruct((NC, CFLAT), jnp.float32))
        scratch += [
            pltpu.VMEM((K, CW), jnp.float32),
            pltpu.VMEM((HRPT, CW), jnp.float32),
            pltpu.VMEM((HRPT * CW,), jnp.float32),
            pltpu.VMEM_SHARED((N, CW), jnp.float32),
        ]

        def body(x_hbm, eidx_hbm, zx_hbm, agg_out, cnt_out,
                 idx_v, rows_v, tidx_v, isem, gsem0, gsem1, ssem, csem,
                 acc_sh, ones_v, cbuf, wbuf, cnt_sh):
            _sc_body(True, K, x_hbm, eidx_hbm, zx_hbm, agg_out, cnt_out,
                     idx_v, rows_v, tidx_v, isem, gsem0, gsem1, ssem, csem,
                     acc_sh, ones_v, cbuf, wbuf, cnt_sh)
    else:
        def body(x_hbm, eidx_hbm, zx_hbm, agg_out,
                 idx_v, rows_v, tidx_v, isem, gsem0, gsem1, ssem, csem,
                 acc_sh):
            _sc_body(False, K, x_hbm, eidx_hbm, zx_hbm, agg_out, None,
                     idx_v, rows_v, tidx_v, isem, gsem0, gsem1, ssem, csem,
                     acc_sh, None, None, None, None)

    return pl.kernel(body, out_type=out_type, mesh=_MESH,
                     scratch_types=scratch,
                     compiler_params=pltpu.CompilerParams(
                         use_tc_tiling_on_sc=False))


_sc_agg_cnt = _make_sc_agg(True, 104)
_sc_agg = _make_sc_agg(False, 96)

BN = 1000


def _tc1_body(agg_ref, cnt_ref, x_ref, wl_ref, bl_ref, wr_ref, o_ref):
    agg = agg_ref[0] + agg_ref[1]
    cnt = cnt_ref[0, :, 0:1] + cnt_ref[1, :, 0:1]
    mean = agg / jnp.maximum(cnt, 1.0)
    h = (jnp.dot(mean, wl_ref[...], preferred_element_type=jnp.float32)
         + bl_ref[...]
         + jnp.dot(x_ref[...], wr_ref[...], preferred_element_type=jnp.float32))
    o_ref[...] = jnp.maximum(h, 0.0)


def _tc2_body(agg_ref, cnt_ref, h_ref, wl_ref, bl_ref, wr_ref,
              wo_ref, bo_ref, o_ref):
    agg = agg_ref[0] + agg_ref[1]
    cnt = cnt_ref[0, :, 0:1] + cnt_ref[1, :, 0:1]
    mean = agg / jnp.maximum(cnt, 1.0)
    h = (jnp.dot(mean, wl_ref[...], preferred_element_type=jnp.float32)
         + bl_ref[...]
         + jnp.dot(h_ref[...], wr_ref[...], preferred_element_type=jnp.float32))
    h = jnp.maximum(h, 0.0)
    o_ref[...] = (jnp.dot(h, wo_ref[...], preferred_element_type=jnp.float32)
                  + bo_ref[...])


def _full_spec(shape):
    return pl.BlockSpec(shape, lambda i: tuple(0 for _ in shape))


_AGG_SPEC = pl.BlockSpec((NC, BN, D), lambda i: (0, i, 0))
_CNT_SPEC = pl.BlockSpec((NC, BN, CW), lambda i: (0, i, 0))
_X_SPEC = pl.BlockSpec((BN, D), lambda i: (i, 0))

_tc1 = pl.pallas_call(
    _tc1_body,
    grid=(N // BN,),
    in_specs=[_AGG_SPEC, _CNT_SPEC, _X_SPEC,
              _full_spec((D, D)), _full_spec((1, D)), _full_spec((D, D))],
    out_specs=_X_SPEC,
    out_shape=jax.ShapeDtypeStruct((N, D), jnp.float32),
)

_tc2 = pl.pallas_call(
    _tc2_body,
    grid=(N // BN,),
    in_specs=[_AGG_SPEC, _CNT_SPEC, _X_SPEC,
              _full_spec((D, D)), _full_spec((1, D)), _full_spec((D, D)),
              _full_spec((D, D)), _full_spec((1, D))],
    out_specs=_X_SPEC,
    out_shape=jax.ShapeDtypeStruct((N, D), jnp.float32),
)


@jax.jit
def kernel(x, edge_index, W_l1, b_l1, W_r1, W_l2, b_l2, W_r2, W_out, b_out):
    eidx = edge_index.astype(jnp.int32)
    zx = jnp.zeros((N, D), jnp.float32)
    agg1, cnt = _sc_agg_cnt(x, eidx, zx)
    cnt = cnt.reshape(NC, N, CW)
    h1 = _tc1(agg1, cnt, x, W_l1, b_l1.reshape(1, D), W_r1)
    agg2 = _sc_agg(h1, eidx, zx)[0]
    return _tc2(agg2, cnt, h1, W_l2, b_l2.reshape(1, D), W_r2,
                W_out, b_out.reshape(1, D))
